# Initial kernel scaffold; baseline (speedup 1.0000x reference)
import jax
import jax.numpy as jnp
from jax.experimental import pallas as pl


def kernel(x, node_weights, node_biases, w1s, b1s, w2s, b2s):
    raise NotImplementedError("write your pallas kernel here")


# speedup vs baseline: 2.3326x; 2.3326x over previous
import functools

import jax
import jax.numpy as jnp
from jax.experimental import pallas as pl
from jax.experimental.pallas import tpu as pltpu

_BT = 512


def _fff_body(x_ref, nw_ref, nb_ref, w1_ref, b1_ref, w2_ref, b2_ref,
              o_ref, m_ref, *, n_leaves):
    l = pl.program_id(1)

    @pl.when(l == 0)
    def _init():
        logits = jnp.dot(x_ref[...], nw_ref[...].T,
                         preferred_element_type=jnp.float32)
        s = jax.nn.sigmoid(logits + nb_ref[...])
        s0 = s[:, 0:1]
        s1 = s[:, 1:2]
        s2 = s[:, 2:3]
        s3 = s[:, 3:4]
        s4 = s[:, 4:5]
        s5 = s[:, 5:6]
        s6 = s[:, 6:7]
        t0 = 1.0 - s0
        t1 = 1.0 - s1
        t2 = 1.0 - s2
        m = jnp.concatenate([
            t0 * t1 * (1.0 - s3), t0 * t1 * s3,
            t0 * s1 * (1.0 - s4), t0 * s1 * s4,
            s0 * t2 * (1.0 - s5), s0 * t2 * s5,
            s0 * s2 * (1.0 - s6), s0 * s2 * s6,
        ], axis=1)
        m_ref[...] = m
        o_ref[...] = jnp.dot(m.astype(jnp.bfloat16), b2_ref[...],
                             preferred_element_type=jnp.float32)

    onehot = (jax.lax.broadcasted_iota(jnp.int32, (1, n_leaves), 1) == l)
    mcol = jnp.sum(m_ref[...] * onehot.astype(jnp.float32),
                   axis=1, keepdims=True)
    h = jnp.dot(x_ref[...], w1_ref[0], preferred_element_type=jnp.float32)
    h = jnp.maximum(h + b1_ref[0], 0.0)
    hs = (h * mcol).astype(jnp.bfloat16)
    o_ref[...] += jnp.dot(hs, w2_ref[0], preferred_element_type=jnp.float32)


def kernel(x, node_weights, node_biases, w1s, b1s, w2s, b2s):
    orig_shape = x.shape
    hidden = x.shape[-1]
    n_leaves, _, leaf = w1s.shape
    x2d = x.reshape(-1, hidden)
    b = x2d.shape[0]
    bt = min(_BT, b)
    pad = (-b) % bt
    if pad:
        x2d = jnp.pad(x2d, ((0, pad), (0, 0)))
    bp = x2d.shape[0]
    n_tb = bp // bt

    xb = x2d.astype(jnp.bfloat16)
    w1b = w1s.astype(jnp.bfloat16)
    w2b = w2s.astype(jnp.bfloat16)
    nwp = jnp.zeros((n_leaves, hidden), jnp.float32).at[:n_leaves - 1].set(
        node_weights).astype(jnp.bfloat16)
    nbp = jnp.zeros((1, n_leaves), jnp.float32).at[0, :n_leaves - 1].set(
        node_biases)
    b2f = b2s.astype(jnp.bfloat16)

    out = pl.pallas_call(
        functools.partial(_fff_body, n_leaves=n_leaves),
        grid=(n_tb, n_leaves),
        in_specs=[
            pl.BlockSpec((bt, hidden), lambda t, l: (t, 0)),
            pl.BlockSpec((n_leaves, hidden), lambda t, l: (0, 0)),
            pl.BlockSpec((1, n_leaves), lambda t, l: (0, 0)),
            pl.BlockSpec((1, hidden, leaf), lambda t, l: (l, 0, 0)),
            pl.BlockSpec((1, 1, leaf), lambda t, l: (l, 0, 0)),
            pl.BlockSpec((1, leaf, hidden), lambda t, l: (l, 0, 0)),
            pl.BlockSpec((n_leaves, hidden), lambda t, l: (0, 0)),
        ],
        out_specs=pl.BlockSpec((bt, hidden), lambda t, l: (t, 0)),
        out_shape=jax.ShapeDtypeStruct((bp, hidden), jnp.float32),
        scratch_shapes=[pltpu.VMEM((bt, n_leaves), jnp.float32)],
    )(xb, nwp, nbp, w1b, b1s.reshape(n_leaves, 1, leaf), w2b, b2f)

    if pad:
        out = out[:b]
    return out.reshape(*orig_shape[:-1], hidden)


# speedup vs baseline: 2.3886x; 1.0240x over previous
import functools

import jax
import jax.numpy as jnp
from jax.experimental import pallas as pl
from jax.experimental.pallas import tpu as pltpu

_BT = 1024


def _fff_body(x_ref, nw_ref, nb_ref, w1_ref, b1_ref, w2_ref, b2_ref,
              o_ref, m_ref, *, n_leaves):
    l = pl.program_id(1)

    @pl.when(l == 0)
    def _init():
        logits = jnp.dot(x_ref[...], nw_ref[...].T,
                         preferred_element_type=jnp.float32)
        s = jax.nn.sigmoid(logits + nb_ref[...])
        s0 = s[:, 0:1]
        s1 = s[:, 1:2]
        s2 = s[:, 2:3]
        s3 = s[:, 3:4]
        s4 = s[:, 4:5]
        s5 = s[:, 5:6]
        s6 = s[:, 6:7]
        t0 = 1.0 - s0
        t1 = 1.0 - s1
        t2 = 1.0 - s2
        m = jnp.concatenate([
            t0 * t1 * (1.0 - s3), t0 * t1 * s3,
            t0 * s1 * (1.0 - s4), t0 * s1 * s4,
            s0 * t2 * (1.0 - s5), s0 * t2 * s5,
            s0 * s2 * (1.0 - s6), s0 * s2 * s6,
        ], axis=1)
        m_ref[...] = m
        o_ref[...] = jnp.dot(m.astype(jnp.bfloat16), b2_ref[...],
                             preferred_element_type=jnp.float32)

    onehot = (jax.lax.broadcasted_iota(jnp.int32, (1, n_leaves), 1) == l)
    mcol = jnp.sum(m_ref[...] * onehot.astype(jnp.float32),
                   axis=1, keepdims=True)
    h = jnp.dot(x_ref[...], w1_ref[0], preferred_element_type=jnp.float32)
    h = jnp.maximum(h + b1_ref[0], 0.0)
    hs = (h * mcol).astype(jnp.bfloat16)
    o_ref[...] += jnp.dot(hs, w2_ref[0], preferred_element_type=jnp.float32)


def kernel(x, node_weights, node_biases, w1s, b1s, w2s, b2s):
    orig_shape = x.shape
    hidden = x.shape[-1]
    n_leaves, _, leaf = w1s.shape
    x2d = x.reshape(-1, hidden)
    b = x2d.shape[0]
    bt = min(_BT, b)
    pad = (-b) % bt
    if pad:
        x2d = jnp.pad(x2d, ((0, pad), (0, 0)))
    bp = x2d.shape[0]
    n_tb = bp // bt

    xb = x2d.astype(jnp.bfloat16)
    w1b = w1s.astype(jnp.bfloat16)
    w2b = w2s.astype(jnp.bfloat16)
    nwp = jnp.zeros((n_leaves, hidden), jnp.float32).at[:n_leaves - 1].set(
        node_weights).astype(jnp.bfloat16)
    nbp = jnp.zeros((1, n_leaves), jnp.float32).at[0, :n_leaves - 1].set(
        node_biases)
    b2f = b2s.astype(jnp.bfloat16)

    out = pl.pallas_call(
        functools.partial(_fff_body, n_leaves=n_leaves),
        grid=(n_tb, n_leaves),
        in_specs=[
            pl.BlockSpec((bt, hidden), lambda t, l: (t, 0)),
            pl.BlockSpec((n_leaves, hidden), lambda t, l: (0, 0)),
            pl.BlockSpec((1, n_leaves), lambda t, l: (0, 0)),
            pl.BlockSpec((1, hidden, leaf), lambda t, l: (l, 0, 0)),
            pl.BlockSpec((1, 1, leaf), lambda t, l: (l, 0, 0)),
            pl.BlockSpec((1, leaf, hidden), lambda t, l: (l, 0, 0)),
            pl.BlockSpec((n_leaves, hidden), lambda t, l: (0, 0)),
        ],
        out_specs=pl.BlockSpec((bt, hidden), lambda t, l: (t, 0)),
        out_shape=jax.ShapeDtypeStruct((bp, hidden), jnp.float32),
        scratch_shapes=[pltpu.VMEM((bt, n_leaves), jnp.float32)],
    )(xb, nwp, nbp, w1b, b1s.reshape(n_leaves, 1, leaf), w2b, b2f)

    if pad:
        out = out[:b]
    return out.reshape(*orig_shape[:-1], hidden)
